# baseline (speedup 1.0000x reference)
import functools

import jax
import jax.numpy as jnp
from jax import lax
from jax.experimental import pallas as pl
from jax.experimental.pallas import tpu as pltpu
from jax.experimental.pallas import tpu_sc as plsc

_F = 128
_K = 65536
_NC = 2
_NS = 16
_NW = _NC * _NS
_CPW = _K // _NW
_C = 256
_NCHUNK = _CPW // _C

_mesh = plsc.VectorSubcoreMesh(core_axis_name="c", subcore_axis_name="s")


@functools.partial(
    pl.kernel,
    out_type=jax.ShapeDtypeStruct((_K, _F), jnp.float32),
    mesh=_mesh,
    scratch_types=[
        pltpu.VMEM((_F, _C), jnp.float32),
        pltpu.VMEM((_F, _C), jnp.float32),
        pltpu.VMEM((_C, _F), jnp.float32),
        pltpu.SemaphoreType.DMA,
        pltpu.SemaphoreType.DMA,
        pltpu.SemaphoreType.DMA,
    ],
    compiler_params=pltpu.CompilerParams(needs_layout_passes=False),
)
def _sc_transpose(q_hbm, out_hbm, in_a, in_b, out_v, sem_ia, sem_ib, sem_o):
    wid = lax.axis_index("s") * _NC + lax.axis_index("c")
    col0 = wid * _CPW
    iota = lax.iota(jnp.int32, 16)
    frows = [f0 + 4 * iota + a for f0 in range(0, _F, 64) for a in range(4)]
    coloffs = [4 * jnp.bitwise_and(iota + d, 15) + b
               for d in range(16) for b in range(4)]

    def _in_slice(ch):
        return q_hbm.at[:, pl.ds(col0 + ch * _C, _C)]

    def _out_slice(ch):
        return out_hbm.at[pl.ds(col0 + ch * _C, _C), :]

    def _permute(in_v):
        for fb in range(_F // 64):

            @plsc.parallel_loop(0, _C // 64)
            def _block(kb):
                k0 = kb * 64
                for db in range(64):
                    kcols = k0 + coloffs[db]
                    for a in range(4):
                        fr = frows[4 * fb + a]
                        v = plsc.load_gather(in_v, [fr, kcols])
                        plsc.store_scatter(out_v, [kcols, fr], v)

    def _half(ch, in_v, sem_i):
        pltpu.make_async_copy(_in_slice(ch), in_v, sem_i).wait()

        @pl.when(ch > 0)
        def _():
            pltpu.make_async_copy(out_v, _out_slice(ch), sem_o).wait()

        _permute(in_v)
        pltpu.async_copy(out_v, _out_slice(ch), sem_o)

        @pl.when(ch + 2 < _NCHUNK)
        def _():
            pltpu.async_copy(_in_slice(ch + 2), in_v, sem_i)

    pltpu.async_copy(_in_slice(0), in_a, sem_ia)
    pltpu.async_copy(_in_slice(1), in_b, sem_ib)

    def _pair(p, carry):
        _half(2 * p, in_a, sem_ia)
        _half(2 * p + 1, in_b, sem_ib)
        return carry

    lax.fori_loop(0, _NCHUNK // 2, _pair, 0)
    pltpu.make_async_copy(out_v, _out_slice(_NCHUNK - 1), sem_o).wait()


def kernel(queue):
    return _sc_transpose(queue)


# speedup vs baseline: 2.2197x; 2.2197x over previous
import functools

import jax
import jax.numpy as jnp
from jax import lax
from jax.experimental import pallas as pl
from jax.experimental.pallas import tpu as pltpu
from jax.experimental.pallas import tpu_sc as plsc

_F = 128
_K = 65536
_NC = 2
_NS = 16
_NW = _NC * _NS
_CPW = _K // _NW
_C = 256
_NCHUNK = _CPW // _C

_mesh = plsc.VectorSubcoreMesh(core_axis_name="c", subcore_axis_name="s")


@functools.partial(
    pl.kernel,
    out_type=jax.ShapeDtypeStruct((_K, _F), jnp.float32),
    mesh=_mesh,
    scratch_types=[
        pltpu.VMEM((_F, _C), jnp.float32),
        pltpu.VMEM((_F, _C), jnp.float32),
        pltpu.VMEM((_C, _F), jnp.float32),
        pltpu.SemaphoreType.DMA,
        pltpu.SemaphoreType.DMA,
        pltpu.SemaphoreType.DMA,
    ],
    compiler_params=pltpu.CompilerParams(needs_layout_passes=False),
)
def _sc_transpose(q_hbm, out_hbm, in_a, in_b, out_v, sem_ia, sem_ib, sem_o):
    wid = lax.axis_index("s") * _NC + lax.axis_index("c")
    col0 = wid * _CPW
    iota = lax.iota(jnp.int32, 16)
    frows = [f0 + 4 * iota + a for f0 in range(0, _F, 64) for a in range(4)]

    def _in_slice(ch):
        return q_hbm.at[:, pl.ds(col0 + ch * _C, _C)]

    def _out_slice(ch):
        return out_hbm.at[pl.ds(col0 + ch * _C, _C), :]

    def _permute(in_v):
        for fb in range(_F // 64):

            @plsc.parallel_loop(0, _C * 64 // 64, unroll=2)
            def _block(t):
                kb = t // 64
                db = t - kb * 64
                d = db // 4
                b = db - d * 4
                kcols = 4 * jnp.bitwise_and(iota + d, 15) + (kb * 64 + b)
                for a in range(4):
                    fr = frows[4 * fb + a]
                    v = plsc.load_gather(in_v, [fr, kcols])
                    plsc.store_scatter(out_v, [kcols, fr], v)

    def _half(ch, in_v, sem_i):
        pltpu.make_async_copy(_in_slice(ch), in_v, sem_i).wait()

        @pl.when(ch > 0)
        def _():
            pltpu.make_async_copy(out_v, _out_slice(ch), sem_o).wait()

        _permute(in_v)
        pltpu.async_copy(out_v, _out_slice(ch), sem_o)

        @pl.when(ch + 2 < _NCHUNK)
        def _():
            pltpu.async_copy(_in_slice(ch + 2), in_v, sem_i)

    pltpu.async_copy(_in_slice(0), in_a, sem_ia)
    pltpu.async_copy(_in_slice(1), in_b, sem_ib)

    def _pair(p, carry):
        _half(2 * p, in_a, sem_ia)
        _half(2 * p + 1, in_b, sem_ib)
        return carry

    lax.fori_loop(0, _NCHUNK // 2, _pair, 0)
    pltpu.make_async_copy(out_v, _out_slice(_NCHUNK - 1), sem_o).wait()


def kernel(queue):
    return _sc_transpose(queue)


# speedup vs baseline: 2.4873x; 1.1205x over previous
import functools

import jax
import jax.numpy as jnp
from jax import lax
from jax.experimental import pallas as pl
from jax.experimental.pallas import tpu as pltpu
from jax.experimental.pallas import tpu_sc as plsc

_F = 128
_K = 65536
_NC = 2
_NS = 16
_NW = _NC * _NS
_CPW = _K // _NW
_C = 128
_NCHUNK = _CPW // _C

_mesh = plsc.VectorSubcoreMesh(core_axis_name="c", subcore_axis_name="s")


@functools.partial(
    pl.kernel,
    out_type=jax.ShapeDtypeStruct((_K, _F), jnp.float32),
    mesh=_mesh,
    scratch_types=[
        pltpu.VMEM((_F, _C), jnp.float32),
        pltpu.VMEM((_F, _C), jnp.float32),
        pltpu.VMEM((_C, _F), jnp.float32),
        pltpu.VMEM((_C, _F), jnp.float32),
        pltpu.SemaphoreType.DMA,
        pltpu.SemaphoreType.DMA,
        pltpu.SemaphoreType.DMA,
        pltpu.SemaphoreType.DMA,
    ],
    compiler_params=pltpu.CompilerParams(needs_layout_passes=False),
)
def _sc_transpose(q_hbm, out_hbm, in_a, in_b, out_a, out_b,
                  sem_ia, sem_ib, sem_oa, sem_ob):
    wid = lax.axis_index("s") * _NC + lax.axis_index("c")
    col0 = wid * _CPW
    iota = lax.iota(jnp.int32, 16)
    frows = [f0 + 4 * iota + a for f0 in range(0, _F, 64) for a in range(4)]

    def _in_slice(ch):
        return q_hbm.at[:, pl.ds(col0 + ch * _C, _C)]

    def _out_slice(ch):
        return out_hbm.at[pl.ds(col0 + ch * _C, _C), :]

    def _permute(in_v, out_v):
        for fb in range(_F // 64):

            @plsc.parallel_loop(0, _C, unroll=2)
            def _block(t):
                kb = t // 64
                db = t - kb * 64
                d = db // 4
                b = db - d * 4
                kcols = 4 * jnp.bitwise_and(iota + d, 15) + (kb * 64 + b)
                for a in range(4):
                    fr = frows[4 * fb + a]
                    v = plsc.load_gather(in_v, [fr, kcols])
                    plsc.store_scatter(out_v, [kcols, fr], v)

    def _half(ch, in_v, out_v, sem_i, sem_o):
        pltpu.make_async_copy(_in_slice(ch), in_v, sem_i).wait()

        @pl.when(ch > 1)
        def _():
            pltpu.make_async_copy(out_v, _out_slice(ch), sem_o).wait()

        _permute(in_v, out_v)
        pltpu.async_copy(out_v, _out_slice(ch), sem_o)

        @pl.when(ch + 2 < _NCHUNK)
        def _():
            pltpu.async_copy(_in_slice(ch + 2), in_v, sem_i)

    pltpu.async_copy(_in_slice(0), in_a, sem_ia)
    pltpu.async_copy(_in_slice(1), in_b, sem_ib)

    def _pair(p, carry):
        _half(2 * p, in_a, out_a, sem_ia, sem_oa)
        _half(2 * p + 1, in_b, out_b, sem_ib, sem_ob)
        return carry

    lax.fori_loop(0, _NCHUNK // 2, _pair, 0)
    pltpu.make_async_copy(out_a, _out_slice(_NCHUNK - 2), sem_oa).wait()
    pltpu.make_async_copy(out_b, _out_slice(_NCHUNK - 1), sem_ob).wait()


def kernel(queue):
    return _sc_transpose(queue)
